# Initial kernel scaffold; baseline (speedup 1.0000x reference)
import jax
import jax.numpy as jnp
from jax.experimental import pallas as pl


def kernel(x, W):
    raise NotImplementedError("write your pallas kernel here")


# speedup vs baseline: 5.7944x; 5.7944x over previous
import jax
import jax.numpy as jnp
from jax.experimental import pallas as pl

_EXPERTS = 64
_TOPK = 2
_BLOCK_T = 2048


def _gate_block(x_ref, w_ref, y_ref, logits_ref):
    x = x_ref[...]
    w = w_ref[...]
    logits = jax.lax.dot_general(
        x, w, (((1,), (1,)), ((), ())), preferred_element_type=jnp.float32
    )
    logits_ref[...] = logits

    m = jnp.max(logits, axis=1, keepdims=True)
    e = jnp.exp(logits - m)
    s = jnp.sum(e, axis=1, keepdims=True)

    col = jax.lax.broadcasted_iota(jnp.int32, logits.shape, 1)
    i1 = jnp.min(jnp.where(logits == m, col, _EXPERTS), axis=1, keepdims=True)
    neg_inf = jnp.float32(-jnp.inf)
    l2 = jnp.where(col == i1, neg_inf, logits)
    m2 = jnp.max(l2, axis=1, keepdims=True)
    i2 = jnp.min(jnp.where(l2 == m2, col, _EXPERTS), axis=1, keepdims=True)
    keep = (col == i1) | (col == i2)
    y_ref[...] = jnp.where(keep, e / s, jnp.float32(0.0))


def kernel(x, W):
    n_tokens, _ = x.shape
    grid = (n_tokens // _BLOCK_T,)
    y, logits = pl.pallas_call(
        _gate_block,
        grid=grid,
        in_specs=[
            pl.BlockSpec((_BLOCK_T, x.shape[1]), lambda i: (i, 0)),
            pl.BlockSpec((W.shape[0], W.shape[1]), lambda i: (0, 0)),
        ],
        out_specs=[
            pl.BlockSpec((_BLOCK_T, _EXPERTS), lambda i: (i, 0)),
            pl.BlockSpec((_BLOCK_T, _EXPERTS), lambda i: (i, 0)),
        ],
        out_shape=[
            jax.ShapeDtypeStruct((n_tokens, _EXPERTS), jnp.float32),
            jax.ShapeDtypeStruct((n_tokens, _EXPERTS), jnp.float32),
        ],
    )(x, W)
    return (y, logits)


# speedup vs baseline: 6.4128x; 1.1067x over previous
import jax
import jax.numpy as jnp
from jax.experimental import pallas as pl

_EXPERTS = 64
_TOPK = 2
_BLOCK_T = 2048


def _gate_block(x_ref, w_ref, y_ref, logits_ref):
    x = x_ref[...]
    w = w_ref[...]
    logits = jax.lax.dot_general(
        x, w, (((1,), (1,)), ((), ())), preferred_element_type=jnp.float32
    )
    logits_ref[...] = logits

    m = jnp.max(logits, axis=1, keepdims=True)
    e = jnp.exp(logits - m)
    s = jnp.sum(e, axis=1, keepdims=True)

    col = jax.lax.broadcasted_iota(jnp.int32, logits.shape, 1).astype(jnp.float32)
    i1 = jnp.min(jnp.where(logits == m, col, jnp.float32(_EXPERTS)),
                 axis=1, keepdims=True)
    at1 = col == i1
    l2 = jnp.where(at1, jnp.float32(-jnp.inf), logits)
    m2 = jnp.max(l2, axis=1, keepdims=True)
    keep = at1 | (l2 == m2)
    y_ref[...] = jnp.where(keep, e / s, jnp.float32(0.0))


def kernel(x, W):
    n_tokens, _ = x.shape
    grid = (n_tokens // _BLOCK_T,)
    y, logits = pl.pallas_call(
        _gate_block,
        grid=grid,
        in_specs=[
            pl.BlockSpec((_BLOCK_T, x.shape[1]), lambda i: (i, 0)),
            pl.BlockSpec((W.shape[0], W.shape[1]), lambda i: (0, 0)),
        ],
        out_specs=[
            pl.BlockSpec((_BLOCK_T, _EXPERTS), lambda i: (i, 0)),
            pl.BlockSpec((_BLOCK_T, _EXPERTS), lambda i: (i, 0)),
        ],
        out_shape=[
            jax.ShapeDtypeStruct((n_tokens, _EXPERTS), jnp.float32),
            jax.ShapeDtypeStruct((n_tokens, _EXPERTS), jnp.float32),
        ],
    )(x, W)
    return (y, logits)


# speedup vs baseline: 6.6148x; 1.0315x over previous
import jax
import jax.numpy as jnp
from jax.experimental import pallas as pl

_EXPERTS = 64
_TOPK = 2
_BLOCK_T = 4096


def _gate_block(x_ref, w_ref, y_ref, logits_ref):
    x = x_ref[...]
    w = w_ref[...]
    logits = jax.lax.dot_general(
        x, w, (((1,), (1,)), ((), ())), preferred_element_type=jnp.float32
    )
    logits_ref[...] = logits

    m = jnp.max(logits, axis=1, keepdims=True)
    e = jnp.exp(logits - m)
    s = jnp.sum(e, axis=1, keepdims=True)

    col = jax.lax.broadcasted_iota(jnp.int32, logits.shape, 1).astype(jnp.float32)
    i1 = jnp.min(jnp.where(logits == m, col, jnp.float32(_EXPERTS)),
                 axis=1, keepdims=True)
    at1 = col == i1
    l2 = jnp.where(at1, jnp.float32(-jnp.inf), logits)
    m2 = jnp.max(l2, axis=1, keepdims=True)
    keep = at1 | (l2 == m2)
    y_ref[...] = jnp.where(keep, e / s, jnp.float32(0.0))


def kernel(x, W):
    n_tokens, _ = x.shape
    grid = (n_tokens // _BLOCK_T,)
    y, logits = pl.pallas_call(
        _gate_block,
        grid=grid,
        in_specs=[
            pl.BlockSpec((_BLOCK_T, x.shape[1]), lambda i: (i, 0)),
            pl.BlockSpec((W.shape[0], W.shape[1]), lambda i: (0, 0)),
        ],
        out_specs=[
            pl.BlockSpec((_BLOCK_T, _EXPERTS), lambda i: (i, 0)),
            pl.BlockSpec((_BLOCK_T, _EXPERTS), lambda i: (i, 0)),
        ],
        out_shape=[
            jax.ShapeDtypeStruct((n_tokens, _EXPERTS), jnp.float32),
            jax.ShapeDtypeStruct((n_tokens, _EXPERTS), jnp.float32),
        ],
    )(x, W)
    return (y, logits)
